# baseline (speedup 1.0000x reference)
import jax
import jax.numpy as jnp
from jax.experimental import pallas as pl
from jax.experimental.pallas import tpu as pltpu

E = 8
K = 2
R = 16
DIN = 2048
DOUT = 2048
ER = E * R
SCALING = 32.0 / 16.0


def _expand_matrix():
    col = jax.lax.broadcasted_iota(jnp.int32, (E, ER), 1)
    row = jax.lax.broadcasted_iota(jnp.int32, (E, ER), 0)
    return (col // R == row).astype(jnp.float32)


def _fused_kernel(x_ref, w_ref, b_ref, a2_ref, wr_ref, bm_ref,
                  out_ref, rw_ref, gate_ref, aw_ref, b3_ref):
    @pl.when(pl.program_id(0) == 0)
    def _prep():
        aw_ref[0:ER, :] = a2_ref[...]
        aw_ref[ER:ER + E, :] = wr_ref[...]
        b3_ref[...] = jnp.transpose(bm_ref[...], (0, 2, 1)).reshape(ER, DOUT)

    xt = x_ref[...]

    haux = jax.lax.dot_general(
        xt, aw_ref[...], (((1,), (1,)), ((), ())),
        preferred_element_type=jnp.float32)
    h = haux[:, :ER]
    gate = haux[:, ER:ER + E]
    gate_ref[...] = gate

    m = jnp.max(gate, axis=-1, keepdims=True)
    p = jnp.exp(gate - m)
    p = p / jnp.sum(p, axis=-1, keepdims=True)

    e_iota = jax.lax.broadcasted_iota(jnp.int32, p.shape, 1)
    m1 = jnp.max(p, axis=-1, keepdims=True)
    i1 = jnp.min(jnp.where(p == m1, e_iota, E), axis=-1, keepdims=True)
    sel1 = e_iota == i1
    p2 = jnp.where(sel1, -jnp.inf, p)
    m2 = jnp.max(p2, axis=-1, keepdims=True)
    i2 = jnp.min(jnp.where(p2 == m2, e_iota, E), axis=-1, keepdims=True)
    sel2 = e_iota == i2
    denom = m1 + m2 + 1e-9
    rw = (jnp.where(sel1, m1, 0.0) + jnp.where(sel2, m2, 0.0)) / denom
    rw_ref[...] = rw

    rw_exp = jax.lax.dot_general(
        rw, _expand_matrix(), (((1,), (0,)), ((), ())),
        preferred_element_type=jnp.float32)
    hp = h * rw_exp * SCALING

    out = jax.lax.dot_general(
        xt, w_ref[...], (((1,), (1,)), ((), ())),
        preferred_element_type=jnp.float32)
    out += b_ref[...]
    out += jax.lax.dot_general(
        hp, b3_ref[...], (((1,), (0,)), ((), ())),
        preferred_element_type=jnp.float32)
    out_ref[...] = out


@jax.jit
def kernel(x, W, b, Wr, A, Bm):
    Bsz, S, _ = x.shape
    N = Bsz * S
    x_flat = x.reshape(N, DIN)
    A2 = A.reshape(ER, DIN)
    b2d = b.reshape(1, DOUT)

    TN = 512
    grid = (N // TN,)

    out, rw, gate = pl.pallas_call(
        _fused_kernel,
        grid=grid,
        in_specs=[
            pl.BlockSpec((TN, DIN), lambda i: (i, 0)),
            pl.BlockSpec((DOUT, DIN), lambda i: (0, 0)),
            pl.BlockSpec((1, DOUT), lambda i: (0, 0)),
            pl.BlockSpec((ER, DIN), lambda i: (0, 0)),
            pl.BlockSpec((E, DIN), lambda i: (0, 0)),
            pl.BlockSpec((E, DOUT, R), lambda i: (0, 0, 0)),
        ],
        out_specs=[
            pl.BlockSpec((TN, DOUT), lambda i: (i, 0)),
            pl.BlockSpec((TN, E), lambda i: (i, 0)),
            pl.BlockSpec((TN, E), lambda i: (i, 0)),
        ],
        out_shape=[
            jax.ShapeDtypeStruct((N, DOUT), jnp.float32),
            jax.ShapeDtypeStruct((N, E), jnp.float32),
            jax.ShapeDtypeStruct((N, E), jnp.float32),
        ],
        scratch_shapes=[
            pltpu.VMEM((ER + E, DIN), jnp.float32),
            pltpu.VMEM((ER, DOUT), jnp.float32),
        ],
        compiler_params=pltpu.CompilerParams(
            dimension_semantics=("arbitrary",)),
    )(x_flat, W, b2d, A2, Wr, Bm)

    return (out.reshape(Bsz, S, DOUT),
            rw.reshape(Bsz, S, E),
            gate.reshape(Bsz, S, E))


# speedup vs baseline: 1.1361x; 1.1361x over previous
import jax
import jax.numpy as jnp
from jax.experimental import pallas as pl
from jax.experimental.pallas import tpu as pltpu

E = 8
K = 2
R = 16
DIN = 2048
DOUT = 2048
ER = E * R
SCALING = 32.0 / 16.0


def _expand_matrix():
    col = jax.lax.broadcasted_iota(jnp.int32, (E, ER), 1)
    row = jax.lax.broadcasted_iota(jnp.int32, (E, ER), 0)
    return (col // R == row).astype(jnp.float32)


def _fused_kernel(x_ref, w_ref, b_ref, aw_ref, b3_ref,
                  out_ref, rw_ref, gate_ref):
    xt = x_ref[...]

    haux = jax.lax.dot_general(
        xt, aw_ref[...], (((1,), (1,)), ((), ())),
        preferred_element_type=jnp.float32)
    h = haux[:, :ER]
    gate = haux[:, ER:ER + E]
    gate_ref[...] = gate

    m = jnp.max(gate, axis=-1, keepdims=True)
    p = jnp.exp(gate - m)
    p = p / jnp.sum(p, axis=-1, keepdims=True)

    e_iota = jax.lax.broadcasted_iota(jnp.int32, p.shape, 1)
    m1 = jnp.max(p, axis=-1, keepdims=True)
    i1 = jnp.min(jnp.where(p == m1, e_iota, E), axis=-1, keepdims=True)
    sel1 = e_iota == i1
    p2 = jnp.where(sel1, -jnp.inf, p)
    m2 = jnp.max(p2, axis=-1, keepdims=True)
    i2 = jnp.min(jnp.where(p2 == m2, e_iota, E), axis=-1, keepdims=True)
    sel2 = e_iota == i2
    denom = m1 + m2 + 1e-9
    rw = (jnp.where(sel1, m1, 0.0) + jnp.where(sel2, m2, 0.0)) / denom
    rw_ref[...] = rw

    rw_exp = jax.lax.dot_general(
        rw, _expand_matrix(), (((1,), (0,)), ((), ())),
        preferred_element_type=jnp.float32)
    hp = h * rw_exp * SCALING

    out = jax.lax.dot_general(
        xt, w_ref[...], (((1,), (1,)), ((), ())),
        preferred_element_type=jnp.float32)
    out += b_ref[...]
    out += jax.lax.dot_general(
        hp, b3_ref[...], (((1,), (0,)), ((), ())),
        preferred_element_type=jnp.float32)
    out_ref[...] = out


@jax.jit
def kernel(x, W, b, Wr, A, Bm):
    Bsz, S, _ = x.shape
    N = Bsz * S
    x_flat = x.reshape(N, DIN)
    A2 = A.reshape(ER, DIN)
    AW = jnp.concatenate([A2, Wr], axis=0)
    B3 = Bm.transpose(0, 2, 1).reshape(ER, DOUT)
    b2d = b.reshape(1, DOUT)

    TN = 512
    grid = (N // TN,)

    out, rw, gate = pl.pallas_call(
        _fused_kernel,
        grid=grid,
        in_specs=[
            pl.BlockSpec((TN, DIN), lambda i: (i, 0)),
            pl.BlockSpec((DOUT, DIN), lambda i: (0, 0)),
            pl.BlockSpec((1, DOUT), lambda i: (0, 0)),
            pl.BlockSpec((ER + E, DIN), lambda i: (0, 0)),
            pl.BlockSpec((ER, DOUT), lambda i: (0, 0)),
        ],
        out_specs=[
            pl.BlockSpec((TN, DOUT), lambda i: (i, 0)),
            pl.BlockSpec((TN, E), lambda i: (i, 0)),
            pl.BlockSpec((TN, E), lambda i: (i, 0)),
        ],
        out_shape=[
            jax.ShapeDtypeStruct((N, DOUT), jnp.float32),
            jax.ShapeDtypeStruct((N, E), jnp.float32),
            jax.ShapeDtypeStruct((N, E), jnp.float32),
        ],
        compiler_params=pltpu.CompilerParams(
            dimension_semantics=("parallel",)),
    )(x_flat, W, b2d, AW, B3)

    return (out.reshape(Bsz, S, DOUT),
            rw.reshape(Bsz, S, E),
            gate.reshape(Bsz, S, E))
